# baseline (speedup 1.0000x reference)
import functools

import jax
import jax.numpy as jnp
from jax.experimental import pallas as pl
from jax.experimental.pallas import tpu as pltpu


def _enc_body(nd, x_ref, pb_ref, w_ref, lb_ref, zpre_ref):
    d = pl.program_id(2)

    @pl.when(d == 0)
    def _init():
        zpre_ref[...] = jnp.zeros_like(zpre_ref)

    xs = x_ref[...] - pb_ref[...]
    zpre_ref[...] += jnp.dot(xs, w_ref[...], preferred_element_type=jnp.float32)

    @pl.when(d == nd - 1)
    def _finish():
        zpre_ref[...] += lb_ref[...]


def _topk_body(K, zpre_ref, z_ref):
    zpre = zpre_ref[...]
    bb = zpre.shape[0]
    lo0 = jnp.zeros((bb, 1), jnp.int32)
    hi0 = jnp.full((bb, 1), 0x7F800000, jnp.int32)
    done0 = jnp.zeros((bb, 1), jnp.int32)

    def cond(carry):
        i, _, _, done = carry
        return jnp.logical_and(i < 31, jnp.sum(done) < done.shape[0])

    def step(carry):
        i, lo, hi, done = carry
        mid = lo + ((hi - lo) >> 1)
        mid_f = jax.lax.bitcast_convert_type(mid, jnp.float32)
        c = jnp.sum((zpre > mid_f).astype(jnp.int32), axis=1, keepdims=True)
        big = c >= K
        act = done == 0
        lo = jnp.where(jnp.logical_and(act, big), mid, lo)
        hi = jnp.where(jnp.logical_and(act, jnp.logical_not(big)), mid, hi)
        done = jnp.maximum(done, (c == K).astype(jnp.int32))
        return i + 1, lo, hi, done

    _, lo, _, _ = jax.lax.while_loop(cond, step, (0, lo0, hi0, done0))
    lo_f = jax.lax.bitcast_convert_type(lo, jnp.float32)
    z_ref[...] = jnp.where(zpre > lo_f, zpre, 0.0)


def _dec_body(nl, z_ref, w_ref, pb_ref, xhat_ref):
    l = pl.program_id(1)

    @pl.when(l == 0)
    def _init():
        xhat_ref[...] = jnp.zeros_like(xhat_ref)

    xhat_ref[...] += jnp.dot(z_ref[...], w_ref[...], preferred_element_type=jnp.float32)

    @pl.when(l == nl - 1)
    def _finish():
        xhat_ref[...] += pb_ref[...]


@jax.jit
def kernel(x_BD, pb_D, W_enc, lb_L, W_dec):
    B, D = x_BD.shape
    L = W_enc.shape[1]
    K = 64

    bb = min(2048, B)
    bl = min(2048, L)
    bd = min(512, D)
    nd = D // bd
    pb2 = pb_D.reshape(1, D)
    lb2 = lb_L.reshape(1, L)

    zpre_BL = pl.pallas_call(
        functools.partial(_enc_body, nd),
        grid=(B // bb, L // bl, nd),
        in_specs=[
            pl.BlockSpec((bb, bd), lambda b, l, d: (b, d)),
            pl.BlockSpec((1, bd), lambda b, l, d: (0, d)),
            pl.BlockSpec((bd, bl), lambda b, l, d: (d, l)),
            pl.BlockSpec((1, bl), lambda b, l, d: (0, l)),
        ],
        out_specs=pl.BlockSpec((bb, bl), lambda b, l, d: (b, l)),
        out_shape=jax.ShapeDtypeStruct((B, L), jnp.float32),
        compiler_params=pltpu.CompilerParams(
            dimension_semantics=("parallel", "parallel", "arbitrary"),
        ),
    )(x_BD, pb2, W_enc, lb2)

    bbm = min(128, B)
    z_BL = pl.pallas_call(
        functools.partial(_topk_body, K),
        grid=(B // bbm,),
        in_specs=[pl.BlockSpec((bbm, L), lambda b: (b, 0))],
        out_specs=pl.BlockSpec((bbm, L), lambda b: (b, 0)),
        out_shape=jax.ShapeDtypeStruct((B, L), jnp.float32),
        compiler_params=pltpu.CompilerParams(
            dimension_semantics=("parallel",),
        ),
    )(zpre_BL)

    bb2 = min(2048, B)
    bl2 = min(512, L)
    nl = L // bl2
    xhat_BD = pl.pallas_call(
        functools.partial(_dec_body, nl),
        grid=(B // bb2, nl),
        in_specs=[
            pl.BlockSpec((bb2, bl2), lambda b, l: (b, l)),
            pl.BlockSpec((bl2, D), lambda b, l: (l, 0)),
            pl.BlockSpec((1, D), lambda b, l: (0, 0)),
        ],
        out_specs=pl.BlockSpec((bb2, D), lambda b, l: (b, 0)),
        out_shape=jax.ShapeDtypeStruct((B, D), jnp.float32),
        compiler_params=pltpu.CompilerParams(
            dimension_semantics=("parallel", "arbitrary"),
        ),
    )(z_BL, W_dec, pb2)

    return (zpre_BL, z_BL, xhat_BD)


# speedup vs baseline: 1.0540x; 1.0540x over previous
import functools

import jax
import jax.numpy as jnp
from jax.experimental import pallas as pl
from jax.experimental.pallas import tpu as pltpu


def _enc_body(nd, x_ref, pb_ref, w_ref, lb_ref, zpre_ref):
    d = pl.program_id(2)

    @pl.when(d == 0)
    def _init():
        zpre_ref[...] = jnp.zeros_like(zpre_ref)

    xs = x_ref[...] - pb_ref[...]
    zpre_ref[...] += jnp.dot(xs, w_ref[...], preferred_element_type=jnp.float32)

    @pl.when(d == nd - 1)
    def _finish():
        zpre_ref[...] += lb_ref[...]


def _topk_body(K, zpre_ref, z_ref):
    zpre = zpre_ref[...]
    bb = zpre.shape[0]
    lo0 = jnp.zeros((bb, 1), jnp.int32)
    hi0 = jnp.full((bb, 1), 0x7F800000, jnp.int32)
    done0 = jnp.zeros((bb, 1), jnp.int32)

    def cond(carry):
        i, _, _, done = carry
        return jnp.logical_and(i < 31, jnp.sum(done) < done.shape[0])

    def step(carry):
        i, lo, hi, done = carry
        mid = lo + ((hi - lo) >> 1)
        mid_f = jax.lax.bitcast_convert_type(mid, jnp.float32)
        c = jnp.sum((zpre > mid_f).astype(jnp.int32), axis=1, keepdims=True)
        big = c >= K
        act = done == 0
        lo = jnp.where(jnp.logical_and(act, big), mid, lo)
        hi = jnp.where(jnp.logical_and(act, jnp.logical_not(big)), mid, hi)
        done = jnp.maximum(done, (c == K).astype(jnp.int32))
        return i + 1, lo, hi, done

    _, lo, _, _ = jax.lax.while_loop(cond, step, (0, lo0, hi0, done0))
    lo_f = jax.lax.bitcast_convert_type(lo, jnp.float32)
    z_ref[...] = jnp.where(zpre > lo_f, zpre, 0.0)


def _dec_body(nl, z_ref, w_ref, pb_ref, xhat_ref):
    l = pl.program_id(1)

    @pl.when(l == 0)
    def _init():
        xhat_ref[...] = jnp.zeros_like(xhat_ref)

    xhat_ref[...] += jnp.dot(z_ref[...], w_ref[...], preferred_element_type=jnp.float32)

    @pl.when(l == nl - 1)
    def _finish():
        xhat_ref[...] += pb_ref[...]


@jax.jit
def kernel(x_BD, pb_D, W_enc, lb_L, W_dec):
    B, D = x_BD.shape
    L = W_enc.shape[1]
    K = 64

    bb = min(2048, B)
    bl = min(2048, L)
    bd = min(512, D)
    nd = D // bd
    pb2 = pb_D.reshape(1, D)
    lb2 = lb_L.reshape(1, L)

    zpre_BL = pl.pallas_call(
        functools.partial(_enc_body, nd),
        grid=(B // bb, L // bl, nd),
        in_specs=[
            pl.BlockSpec((bb, bd), lambda b, l, d: (b, d)),
            pl.BlockSpec((1, bd), lambda b, l, d: (0, d)),
            pl.BlockSpec((bd, bl), lambda b, l, d: (d, l)),
            pl.BlockSpec((1, bl), lambda b, l, d: (0, l)),
        ],
        out_specs=pl.BlockSpec((bb, bl), lambda b, l, d: (b, l)),
        out_shape=jax.ShapeDtypeStruct((B, L), jnp.float32),
        compiler_params=pltpu.CompilerParams(
            dimension_semantics=("parallel", "parallel", "arbitrary"),
        ),
    )(x_BD, pb2, W_enc, lb2)

    bbm = min(256, B)
    z_BL = pl.pallas_call(
        functools.partial(_topk_body, K),
        grid=(B // bbm,),
        in_specs=[pl.BlockSpec((bbm, L), lambda b: (b, 0))],
        out_specs=pl.BlockSpec((bbm, L), lambda b: (b, 0)),
        out_shape=jax.ShapeDtypeStruct((B, L), jnp.float32),
        compiler_params=pltpu.CompilerParams(
            dimension_semantics=("parallel",),
        ),
    )(zpre_BL)

    bb2 = min(2048, B)
    bl2 = min(512, L)
    nl = L // bl2
    xhat_BD = pl.pallas_call(
        functools.partial(_dec_body, nl),
        grid=(B // bb2, nl),
        in_specs=[
            pl.BlockSpec((bb2, bl2), lambda b, l: (b, l)),
            pl.BlockSpec((bl2, D), lambda b, l: (l, 0)),
            pl.BlockSpec((1, D), lambda b, l: (0, 0)),
        ],
        out_specs=pl.BlockSpec((bb2, D), lambda b, l: (b, 0)),
        out_shape=jax.ShapeDtypeStruct((B, D), jnp.float32),
        compiler_params=pltpu.CompilerParams(
            dimension_semantics=("parallel", "arbitrary"),
        ),
    )(z_BL, W_dec, pb2)

    return (zpre_BL, z_BL, xhat_BD)
